# baseline (speedup 1.0000x reference)
import jax
import jax.numpy as jnp
from jax import lax
from jax.experimental import pallas as pl
from jax.experimental.pallas import tpu as pltpu
from jax.experimental.pallas import tpu_sc as plsc

B = 16384
NUM_MASH_STEP_TYPES = 16
NUM_HOP_STAGE = 8
NUM_MISC_STAGE = 8
NUM_MO_STAGE = 4

OUT_W = 2663
PAD_W8 = 333

NC, NS = 2, 16
NW = NC * NS
RPW = B // NW
R = 32
NCHUNK = RPW // R
L = 16

TB_G, TB_A, TB_H, TB_M, TB_O = 0, 1024, 1280, 3328, 3840
TBL_ROWS = 4352

E_G, E_A, E_H, E_M, E_O = 0, 64, 80, 208, 240

O_HEAD = (0, 10)
O_GEMB, O_GAMT = (10, 64), (74, 2)
O_AEMB, O_AAMT = (76, 16), (92, 1)
O_HEMB, O_HOH, O_HT, O_HC = (93, 128), (221, 32), (253, 4), (257, 4)
O_MEMB, O_MOH, O_MT, O_MA = (261, 32), (293, 16), (309, 2), (311, 2)
O_OEMB, O_OOH = (313, 16), (329, 4)


RB = 256


def _unshift_body(src_ref, dst_ref):
  dst_ref[...] = src_ref[:, 1:]


def _unshift(padded2d):
  return pl.pallas_call(
      _unshift_body,
      grid=(B // RB,),
      in_specs=[pl.BlockSpec((RB, PAD_W8 * 8), lambda i: (i, 0))],
      out_specs=pl.BlockSpec((RB, OUT_W), lambda i: (i, 0)),
      out_shape=jax.ShapeDtypeStruct((B, OUT_W), jnp.float32),
  )(padded2d)


def _worker_id():
  return lax.axis_index("s") * NC + lax.axis_index("c")


def _vgather1(ref, rows):
  return plsc.load_gather(ref, [rows])


def _vgather2(ref, rows, cols):
  return plsc.load_gather(ref, [rows, cols])


def _vscatter3(ref, i0, i1, i2, x):
  plsc.store_scatter(ref, [i0, i1, i2], x)


def _gather_start(tbl, idx_row, dst, sem):
  return pltpu.async_copy(tbl.at[idx_row], dst, sem)


def _body(
    boil, ph, sp, mtyp, mtime, mtemp, ftime, ftemp,
    idx_all, tbl8,
    gamt, aamt, htime, hconc, mtime2, mamt,
    hstg, mstg, ostg,
    out,
    idxv, emb3, head, h_oh, m_oh, o_oh,
    boil_v, ph_v, sp_v, mtyp_v, mtime_v, mtemp_v, ftime_v, ftemp_v,
    hstg_v, mstg_v, ostg_v,
    gamt_v, aamt_v, htime_v, hconc_v, mtime2_v, mamt_v,
    sem,
):
  wid = _worker_id()
  iota = lax.iota(jnp.int32, L)
  ones = jnp.full((L,), 1.0, jnp.float32)
  zeros = jnp.zeros((L,), jnp.float32)

  def zero_buf(buf, w8):
    def zb(c, _):
      for g in range(R // L):
        cc = iota * 0 + c
        _vscatter3(buf, iota + g * L, lax.shift_right_logical(cc, 3),
                   lax.bitwise_and(cc, 7), zeros)
      return 0
    lax.fori_loop(0, w8 * 8, zb, 0)

  zero_buf(head, O_HEAD[1])
  zero_buf(h_oh, O_HOH[1])
  zero_buf(m_oh, O_MOH[1])
  zero_buf(o_oh, O_OOH[1])

  def chunk(c, _):
    row0 = wid * RPW + c * R
    rsl = pl.ds(row0, R)

    idx_desc = pltpu.async_copy(idx_all.at[rsl], idxv, sem)
    copies = [
        (boil.at[rsl], boil_v),
        (ph.at[rsl], ph_v),
        (sp.at[rsl], sp_v),
        (mtyp.at[rsl], mtyp_v),
        (mtime.at[rsl], mtime_v),
        (mtemp.at[rsl], mtemp_v),
        (ftime.at[rsl], ftime_v),
        (ftemp.at[rsl], ftemp_v),
        (hstg.at[rsl], hstg_v),
        (mstg.at[rsl], mstg_v),
        (ostg.at[rsl], ostg_v),
        (gamt.at[rsl], gamt_v),
        (aamt.at[rsl], aamt_v),
        (htime.at[rsl], htime_v),
        (hconc.at[rsl], hconc_v),
        (mtime2.at[rsl], mtime2_v),
        (mamt.at[rsl], mamt_v),
    ]
    descs = [pltpu.async_copy(s, d, sem) for s, d in copies]
    idx_desc.wait()

    gathers = []
    for r in range(R):
      for m in range(2):
        gathers.append(_gather_start(
            tbl8, idxv.at[r, m], emb3.at[r, pl.ds(m * 128, 128), :], sem))

    for d in descs:
      d.wait()

    scatters = []

    def put_col(src_v, j, buf, col):
      for g in range(R // L):
        rows = iota + g * L
        if j is None:
          val = _vgather1(src_v, rows)
        else:
          val = _vgather2(src_v, rows, iota * 0 + j)
        _vscatter3(buf, rows, iota * 0 + (col >> 3), iota * 0 + (col & 7), val)

    def put_ones(stg_v, j, buf, i1_of_t, i2_of_t):
      for g in range(R // L):
        rows = iota + g * L
        t = _vgather2(stg_v, rows, iota * 0 + j)
        i1, i2 = i1_of_t(t), i2_of_t(t)
        _vscatter3(buf, rows, i1, i2, ones)
        scatters.append((buf, rows, i1, i2))

    put_col(boil_v, None, head, 1)
    put_col(ph_v, None, head, 2)
    put_col(sp_v, None, head, 3)
    for j in range(4):
      base = 4 + 16 * j
      put_ones(mtyp_v, j, head,
               lambda t, base=base: lax.shift_right_logical(base + t, 3),
               lambda t, base=base: lax.bitwise_and(base + t, 7))
      put_col(mtime_v, j, head, 68 + j)
      put_col(mtemp_v, j, head, 72 + j)
    for j in range(2):
      put_col(ftime_v, j, head, 76 + j)
      put_col(ftemp_v, j, head, 78 + j)
    for j in range(32):
      put_ones(hstg_v, j, h_oh, lambda t, j=j: iota * 0 + j, lambda t: t)
    for j in range(16):
      put_ones(mstg_v, j, m_oh, lambda t, j=j: iota * 0 + j, lambda t: t)
    for j in range(8):
      put_ones(ostg_v, j, o_oh, lambda t, j=j: iota * 0 + (j >> 1),
               lambda t, j=j: ((j & 1) << 2) + t)

    for d in gathers:
      d.wait()

    def osl(off_w):
      return out.at[rsl, pl.ds(off_w[0], off_w[1]), :]

    def esl(e0, w8):
      return emb3.at[:, pl.ds(e0, w8), :]

    outs = [
        (head, osl(O_HEAD)),
        (esl(E_G, 64), osl(O_GEMB)),
        (esl(E_A, 16), osl(O_AEMB)),
        (esl(E_H, 128), osl(O_HEMB)),
        (h_oh, osl(O_HOH)),
        (esl(E_M, 32), osl(O_MEMB)),
        (m_oh, osl(O_MOH)),
        (esl(E_O, 16), osl(O_OEMB)),
        (o_oh, osl(O_OOH)),
    ]
    for buf2d, (o0, n8) in [
        (gamt_v, O_GAMT), (aamt_v, O_AAMT), (htime_v, O_HT),
        (hconc_v, O_HC), (mtime2_v, O_MT), (mamt_v, O_MA),
    ]:
      for g in range(n8):
        outs.append((buf2d.at[:, pl.ds(8 * g, 8)], out.at[rsl, o0 + g, :]))
    descs2 = [pltpu.async_copy(s, d, sem) for s, d in outs]
    for d in descs2:
      d.wait()

    for buf, rows, i1, i2 in scatters:
      _vscatter3(buf, rows, i1, i2, zeros)
    return 0

  lax.fori_loop(0, NCHUNK, chunk, 0)


@jax.jit
def kernel(boil_time, mash_ph, sparge_temp, mash_step_type_inds,
           mash_step_times, mash_step_avg_temps, ferment_stage_times,
           ferment_stage_temps, grain_core_type_inds, grain_amts,
           adjunct_core_type_inds, adjunct_amts, hop_type_inds,
           hop_stage_type_inds, hop_times, hop_concentrations,
           misc_type_inds, misc_stage_inds, misc_times, misc_amts,
           mo_type_inds, mo_stage_inds, W_grain, W_adjunct, W_hop, W_misc,
           W_mo):
  f32, i32 = jnp.float32, jnp.int32

  tbl8 = jnp.concatenate([
      W_grain.reshape(-1, 8), W_adjunct.reshape(-1, 8),
      W_hop.reshape(-1, 8), W_misc.reshape(-1, 8), W_mo.reshape(-1, 8),
  ], axis=0)

  def sub(idx, base, n):
    return (base + idx[:, :, None] * n + jnp.arange(n, dtype=i32)).reshape(B, -1)

  idx_all = jnp.concatenate([
      sub(grain_core_type_inds, TB_G, 4),
      sub(adjunct_core_type_inds, TB_A, 2),
      sub(hop_type_inds, TB_H, 4),
      sub(misc_type_inds, TB_M, 2),
      sub(mo_type_inds, TB_O, 2),
  ], axis=1).reshape(B, 2, 128)

  mesh = plsc.VectorSubcoreMesh(core_axis_name="c", subcore_axis_name="s",
                                num_cores=NC, num_subcores=NS)
  run = pl.kernel(
      _body,
      out_type=jax.ShapeDtypeStruct((B, PAD_W8, 8), f32),
      mesh=mesh,
      compiler_params=pltpu.CompilerParams(use_tc_tiling_on_sc=False,
                                           needs_layout_passes=False),
      scratch_types=[
          pltpu.VMEM((R, 2, 128), i32),
          pltpu.VMEM((R, 256, 8), f32),
          pltpu.VMEM((R, O_HEAD[1], 8), f32),
          pltpu.VMEM((R, O_HOH[1], 8), f32),
          pltpu.VMEM((R, O_MOH[1], 8), f32),
          pltpu.VMEM((R, O_OOH[1], 8), f32),
          pltpu.VMEM((R,), f32),
          pltpu.VMEM((R,), f32),
          pltpu.VMEM((R,), f32),
          pltpu.VMEM((R, 4), i32),
          pltpu.VMEM((R, 4), f32),
          pltpu.VMEM((R, 4), f32),
          pltpu.VMEM((R, 2), f32),
          pltpu.VMEM((R, 2), f32),
          pltpu.VMEM((R, 32), i32),
          pltpu.VMEM((R, 16), i32),
          pltpu.VMEM((R, 8), i32),
          pltpu.VMEM((R, 16), f32),
          pltpu.VMEM((R, 8), f32),
          pltpu.VMEM((R, 32), f32),
          pltpu.VMEM((R, 32), f32),
          pltpu.VMEM((R, 16), f32),
          pltpu.VMEM((R, 16), f32),
          pltpu.SemaphoreType.DMA,
      ],
  )
  padded = run(
      boil_time, mash_ph, sparge_temp,
      mash_step_type_inds,
      mash_step_times, mash_step_avg_temps,
      ferment_stage_times, ferment_stage_temps,
      idx_all, tbl8,
      grain_amts, adjunct_amts, hop_times, hop_concentrations,
      misc_times, misc_amts,
      hop_stage_type_inds, misc_stage_inds, mo_stage_inds,
  )
  return _unshift(padded.reshape(B, PAD_W8 * 8))


# speedup vs baseline: 3.1837x; 3.1837x over previous
import jax
import jax.numpy as jnp
from jax import lax
from jax.experimental import pallas as pl
from jax.experimental.pallas import tpu as pltpu
from jax.experimental.pallas import tpu_sc as plsc

B = 16384
NUM_MASH_STEP_TYPES = 16
NUM_HOP_STAGE = 8
NUM_MISC_STAGE = 8
NUM_MO_STAGE = 4

OUT_W = 2663

NC, NS = 2, 16
NW = NC * NS
RPW = B // NW
R = 32
NCHUNK = RPW // R
L = 16

TB_G, TB_A, TB_H, TB_M, TB_O = 0, 1024, 1280, 3328, 3840
TBL_ROWS = 4352

S_GAMT, S_AAMT, S_HT, S_HC = 80, 96, 104, 136
S_MT, S_MA, S_OOH = 168, 184, 200
SMALL_W = 256

RB = 512


def _worker_id():
  return lax.axis_index("s") * NC + lax.axis_index("c")


def _vgather1(ref, rows):
  return plsc.load_gather(ref, [rows])


def _vgather2(ref, rows, cols):
  return plsc.load_gather(ref, [rows, cols])


def _vscatter2(ref, i0, i1, x):
  plsc.store_scatter(ref, [i0, i1], x)


def _gather_start(tbl, idx_row, dst, sem):
  return pltpu.async_copy(tbl.at[idx_row], dst, sem)


def _body(
    boil, ph, sp, mtyp, mtime, mtemp, ftime, ftemp,
    gidx, aidx, hidx, midx, oidx, tbl8,
    gamt, aamt, htime, hconc, mtime2, mamt,
    hstg, mstg, ostg,
    small_o, gemb_o, aemb_o, hemb_o, memb_o, oemb_o, hoh_o, moh_o,
    gidx_v, aidx_v, hidx_v, midx_v, oidx_v,
    gemb_v, aemb_v, hemb_v, memb_v, oemb_v,
    small_v, hoh_v, moh_v,
    boil_v, ph_v, sp_v, mtyp_v, mtime_v, mtemp_v, ftime_v, ftemp_v,
    hstg_v, mstg_v, ostg_v,
    gamt_v, aamt_v, htime_v, hconc_v, mtime2_v, mamt_v,
    sem,
):
  wid = _worker_id()
  iota = lax.iota(jnp.int32, L)
  ones = jnp.full((L,), 1.0, jnp.float32)
  zeros = jnp.zeros((L,), jnp.float32)

  def zero_buf(buf, nsub):
    def zb(q, _):
      for g in range(R // L):
        qq = iota * 0 + q
        _vscatter2(buf, (iota + g * L) * nsub + (q >> 3), qq & 7, zeros)
      return 0
    lax.fori_loop(0, nsub * 8, zb, 0)

  zero_buf(small_v, SMALL_W // 8)
  zero_buf(hoh_v, 32)
  zero_buf(moh_v, 16)

  def chunk(c, _):
    row0 = wid * RPW + c * R
    rsl = pl.ds(row0, R)

    idx_descs = [
        pltpu.async_copy(gidx.at[pl.ds(row0 // 2, R * 64 // 128)], gidx_v, sem),
        pltpu.async_copy(aidx.at[pl.ds(row0 // 8, R * 16 // 128)], aidx_v, sem),
        pltpu.async_copy(hidx.at[pl.ds(row0, R)], hidx_v, sem),
        pltpu.async_copy(midx.at[pl.ds(row0 // 4, R * 32 // 128)], midx_v, sem),
        pltpu.async_copy(oidx.at[pl.ds(row0 // 8, R * 16 // 128)], oidx_v, sem),
    ]
    copies = [
        (boil.at[rsl], boil_v),
        (ph.at[rsl], ph_v),
        (sp.at[rsl], sp_v),
        (mtyp.at[rsl], mtyp_v),
        (mtime.at[rsl], mtime_v),
        (mtemp.at[rsl], mtemp_v),
        (ftime.at[rsl], ftime_v),
        (ftemp.at[rsl], ftemp_v),
        (hstg.at[rsl], hstg_v),
        (mstg.at[rsl], mstg_v),
        (ostg.at[rsl], ostg_v),
        (gamt.at[rsl], gamt_v),
        (aamt.at[rsl], aamt_v),
        (htime.at[rsl], htime_v),
        (hconc.at[rsl], hconc_v),
        (mtime2.at[rsl], mtime2_v),
        (mamt.at[rsl], mamt_v),
    ]
    descs = [pltpu.async_copy(s, d, sem) for s, d in copies]
    for d in idx_descs:
      d.wait()

    gathers = []
    for idxb, embb, n in (
        (gidx_v, gemb_v, R * 64 // 128),
        (aidx_v, aemb_v, R * 16 // 128),
        (hidx_v, hemb_v, R * 128 // 128),
        (midx_v, memb_v, R * 32 // 128),
        (oidx_v, oemb_v, R * 16 // 128),
    ):
      for j in range(n):
        gathers.append(_gather_start(
            tbl8, idxb.at[j], embb.at[pl.ds(j * 128, 128)], sem))

    for d in descs:
      d.wait()

    scatters = []

    def put_small(src_v, j, col):
      for g in range(R // L):
        rows = iota + g * L
        if j is None:
          val = _vgather1(src_v, rows)
        else:
          val = _vgather2(src_v, rows, iota * 0 + j)
        _vscatter2(small_v, rows * (SMALL_W // 8) + (col >> 3),
                   iota * 0 + (col & 7), val)

    def put_ones(stg_v, j, buf, nsub, cof, k):
      for g in range(R // L):
        rows = iota + g * L
        t = _vgather2(stg_v, rows, iota * 0 + j)
        cv = cof + j * k + t
        i0 = rows * nsub + lax.shift_right_logical(cv, 3)
        i1 = lax.bitwise_and(cv, 7)
        _vscatter2(buf, i0, i1, ones)
        scatters.append((buf, i0, i1))

    put_small(boil_v, None, 0)
    put_small(ph_v, None, 1)
    put_small(sp_v, None, 2)
    for j in range(4):
      put_ones(mtyp_v, j, small_v, SMALL_W // 8, 3, NUM_MASH_STEP_TYPES)
      put_small(mtime_v, j, 67 + j)
      put_small(mtemp_v, j, 71 + j)
    for j in range(2):
      put_small(ftime_v, j, 75 + j)
      put_small(ftemp_v, j, 77 + j)
    for src_v, w, cof in ((gamt_v, 16, S_GAMT), (aamt_v, 8, S_AAMT),
                          (htime_v, 32, S_HT), (hconc_v, 32, S_HC),
                          (mtime2_v, 16, S_MT), (mamt_v, 16, S_MA)):
      for j in range(w):
        put_small(src_v, j, cof + j)
    for j in range(32):
      put_ones(hstg_v, j, hoh_v, 32, 0, NUM_HOP_STAGE)
    for j in range(16):
      put_ones(mstg_v, j, moh_v, 16, 0, NUM_MISC_STAGE)
    for j in range(8):
      put_ones(ostg_v, j, small_v, SMALL_W // 8, S_OOH, NUM_MO_STAGE)

    for d in gathers:
      d.wait()

    outs = [
        (small_v, small_o.at[pl.ds(row0 * 32, R * 32)]),
        (gemb_v, gemb_o.at[pl.ds(row0 * 64, R * 64)]),
        (aemb_v, aemb_o.at[pl.ds(row0 * 16, R * 16)]),
        (hemb_v, hemb_o.at[pl.ds(row0 * 128, R * 128)]),
        (memb_v, memb_o.at[pl.ds(row0 * 32, R * 32)]),
        (oemb_v, oemb_o.at[pl.ds(row0 * 16, R * 16)]),
        (hoh_v, hoh_o.at[pl.ds(row0 * 32, R * 32)]),
        (moh_v, moh_o.at[pl.ds(row0 * 16, R * 16)]),
    ]
    descs2 = [pltpu.async_copy(s, d, sem) for s, d in outs]
    for d in descs2:
      d.wait()

    for buf, i0, i1 in scatters:
      _vscatter2(buf, i0, i1, zeros)
    return 0

  lax.fori_loop(0, NCHUNK, chunk, 0)


def _concat_body(small, gemb, aemb, hemb, memb, oemb, hoh, moh, out):
  out[...] = jnp.concatenate([
      small[:, 0:79],
      gemb[...],
      small[:, S_GAMT:S_GAMT + 16],
      aemb[...],
      small[:, S_AAMT:S_AAMT + 8],
      hemb[...],
      hoh[...],
      small[:, S_HT:S_HT + 32],
      small[:, S_HC:S_HC + 32],
      memb[...],
      moh[...],
      small[:, S_MT:S_MT + 16],
      small[:, S_MA:S_MA + 16],
      oemb[...],
      small[:, S_OOH:S_OOH + 32],
  ], axis=1)


def _assemble(small2, gemb2, aemb2, hemb2, memb2, oemb2, hoh2, moh2):
  def spec(w):
    return pl.BlockSpec((RB, w), lambda i: (i, 0))
  return pl.pallas_call(
      _concat_body,
      grid=(B // RB,),
      in_specs=[spec(SMALL_W), spec(512), spec(128), spec(1024),
                spec(256), spec(128), spec(256), spec(128)],
      out_specs=pl.BlockSpec((RB, OUT_W), lambda i: (i, 0)),
      out_shape=jax.ShapeDtypeStruct((B, OUT_W), jnp.float32),
  )(small2, gemb2, aemb2, hemb2, memb2, oemb2, hoh2, moh2)


@jax.jit
def kernel(boil_time, mash_ph, sparge_temp, mash_step_type_inds,
           mash_step_times, mash_step_avg_temps, ferment_stage_times,
           ferment_stage_temps, grain_core_type_inds, grain_amts,
           adjunct_core_type_inds, adjunct_amts, hop_type_inds,
           hop_stage_type_inds, hop_times, hop_concentrations,
           misc_type_inds, misc_stage_inds, misc_times, misc_amts,
           mo_type_inds, mo_stage_inds, W_grain, W_adjunct, W_hop, W_misc,
           W_mo):
  f32, i32 = jnp.float32, jnp.int32

  tbl8 = jnp.concatenate([
      W_grain.reshape(-1, 8), W_adjunct.reshape(-1, 8),
      W_hop.reshape(-1, 8), W_misc.reshape(-1, 8), W_mo.reshape(-1, 8),
  ], axis=0)

  def sub(idx, base, n):
    e = (base + idx[:, :, None] * n + jnp.arange(n, dtype=i32))
    return e.reshape(-1, 128)

  gidx = sub(grain_core_type_inds, TB_G, 4)
  aidx = sub(adjunct_core_type_inds, TB_A, 2)
  hidx = sub(hop_type_inds, TB_H, 4)
  midx = sub(misc_type_inds, TB_M, 2)
  oidx = sub(mo_type_inds, TB_O, 2)

  mesh = plsc.VectorSubcoreMesh(core_axis_name="c", subcore_axis_name="s",
                                num_cores=NC, num_subcores=NS)
  run = pl.kernel(
      _body,
      out_type=[
          jax.ShapeDtypeStruct((B * 32, 8), f32),
          jax.ShapeDtypeStruct((B * 64, 8), f32),
          jax.ShapeDtypeStruct((B * 16, 8), f32),
          jax.ShapeDtypeStruct((B * 128, 8), f32),
          jax.ShapeDtypeStruct((B * 32, 8), f32),
          jax.ShapeDtypeStruct((B * 16, 8), f32),
          jax.ShapeDtypeStruct((B * 32, 8), f32),
          jax.ShapeDtypeStruct((B * 16, 8), f32),
      ],
      mesh=mesh,
      compiler_params=pltpu.CompilerParams(use_tc_tiling_on_sc=False,
                                           needs_layout_passes=False),
      scratch_types=[
          pltpu.VMEM((R * 64 // 128, 128), i32),
          pltpu.VMEM((R * 16 // 128, 128), i32),
          pltpu.VMEM((R, 128), i32),
          pltpu.VMEM((R * 32 // 128, 128), i32),
          pltpu.VMEM((R * 16 // 128, 128), i32),
          pltpu.VMEM((R * 64, 8), f32),
          pltpu.VMEM((R * 16, 8), f32),
          pltpu.VMEM((R * 128, 8), f32),
          pltpu.VMEM((R * 32, 8), f32),
          pltpu.VMEM((R * 16, 8), f32),
          pltpu.VMEM((R * 32, 8), f32),
          pltpu.VMEM((R * 32, 8), f32),
          pltpu.VMEM((R * 16, 8), f32),
          pltpu.VMEM((R,), f32),
          pltpu.VMEM((R,), f32),
          pltpu.VMEM((R,), f32),
          pltpu.VMEM((R, 4), i32),
          pltpu.VMEM((R, 4), f32),
          pltpu.VMEM((R, 4), f32),
          pltpu.VMEM((R, 2), f32),
          pltpu.VMEM((R, 2), f32),
          pltpu.VMEM((R, 32), i32),
          pltpu.VMEM((R, 16), i32),
          pltpu.VMEM((R, 8), i32),
          pltpu.VMEM((R, 16), f32),
          pltpu.VMEM((R, 8), f32),
          pltpu.VMEM((R, 32), f32),
          pltpu.VMEM((R, 32), f32),
          pltpu.VMEM((R, 16), f32),
          pltpu.VMEM((R, 16), f32),
          pltpu.SemaphoreType.DMA,
      ],
  )
  small_o, gemb_o, aemb_o, hemb_o, memb_o, oemb_o, hoh_o, moh_o = run(
      boil_time, mash_ph, sparge_temp,
      mash_step_type_inds,
      mash_step_times, mash_step_avg_temps,
      ferment_stage_times, ferment_stage_temps,
      gidx, aidx, hidx, midx, oidx, tbl8,
      grain_amts, adjunct_amts, hop_times, hop_concentrations,
      misc_times, misc_amts,
      hop_stage_type_inds, misc_stage_inds, mo_stage_inds,
  )
  return _assemble(
      small_o.reshape(B, SMALL_W), gemb_o.reshape(B, 512),
      aemb_o.reshape(B, 128), hemb_o.reshape(B, 1024),
      memb_o.reshape(B, 256), oemb_o.reshape(B, 128),
      hoh_o.reshape(B, 256), moh_o.reshape(B, 128),
  )


# speedup vs baseline: 3.5575x; 1.1174x over previous
import jax
import jax.numpy as jnp
from jax import lax
from jax.experimental import pallas as pl
from jax.experimental.pallas import tpu as pltpu
from jax.experimental.pallas import tpu_sc as plsc

B = 16384
NUM_MASH_STEP_TYPES = 16
NUM_HOP_STAGE = 8
NUM_MISC_STAGE = 8
NUM_MO_STAGE = 4

OUT_W = 2663

NC, NS = 2, 16
NW = NC * NS
RPW = B // NW
R = 32
NCHUNK = RPW // R
L = 16

TB_G, TB_A, TB_H, TB_M, TB_O = 0, 1024, 1280, 3328, 3840
TBL_ROWS = 4352

S_GAMT, S_AAMT, S_HT, S_HC = 80, 96, 104, 136
S_MT, S_MA, S_OOH = 168, 184, 200
SMALL_W = 256

RB = 512


def _worker_id():
  return lax.axis_index("s") * NC + lax.axis_index("c")


def _vgather1(ref, rows):
  return plsc.load_gather(ref, [rows])


def _vgather2(ref, rows, cols):
  return plsc.load_gather(ref, [rows, cols])


def _vscatter2(ref, i0, i1, x):
  plsc.store_scatter(ref, [i0, i1], x)


def _gather_start(tbl, idx_row, dst, sem):
  return pltpu.async_copy(tbl.at[idx_row], dst, sem)


def _body(
    boil, ph, sp, mtyp, mtime, mtemp, ftime, ftemp,
    gidx, aidx, hidx, midx, oidx, tbl8,
    gamt, aamt, htime, hconc, mtime2, mamt,
    hstg, mstg, ostg,
    small_o, gemb_o, aemb_o, hemb_o, memb_o, oemb_o, hoh_o, moh_o,
    gidx_v, aidx_v, hidx_v, midx_v, oidx_v,
    gemb_v, aemb_v, hemb_v, memb_v, oemb_v,
    small_v, hoh_v, moh_v,
    boil_v, ph_v, sp_v, mtyp_v, mtime_v, mtemp_v, ftime_v, ftemp_v,
    hstg_v, mstg_v, ostg_v,
    gamt_v, aamt_v, htime_v, hconc_v, mtime2_v, mamt_v,
    sem,
):
  wid = _worker_id()
  iota = lax.iota(jnp.int32, L)
  ones = jnp.full((L,), 1.0, jnp.float32)
  zeros = jnp.zeros((L,), jnp.float32)

  def zero_buf(buf, nsub):
    def zb(q, _):
      for g in range(R // L):
        qq = iota * 0 + q
        _vscatter2(buf, (iota + g * L) * nsub + (q >> 3), qq & 7, zeros)
      return 0
    lax.fori_loop(0, nsub * 8, zb, 0)

  zero_buf(small_v, SMALL_W // 8)
  zero_buf(hoh_v, 32)
  zero_buf(moh_v, 16)

  def chunk(c, _):
    row0 = wid * RPW + c * R
    rsl = pl.ds(row0, R)

    idx_descs = [
        pltpu.async_copy(gidx.at[pl.ds(row0 // 2, R * 64 // 128)], gidx_v, sem),
        pltpu.async_copy(aidx.at[pl.ds(row0 // 8, R * 16 // 128)], aidx_v, sem),
        pltpu.async_copy(hidx.at[pl.ds(row0, R)], hidx_v, sem),
        pltpu.async_copy(midx.at[pl.ds(row0 // 4, R * 32 // 128)], midx_v, sem),
        pltpu.async_copy(oidx.at[pl.ds(row0 // 8, R * 16 // 128)], oidx_v, sem),
    ]
    copies = [
        (boil.at[rsl], boil_v),
        (ph.at[rsl], ph_v),
        (sp.at[rsl], sp_v),
        (mtyp.at[rsl], mtyp_v),
        (mtime.at[rsl], mtime_v),
        (mtemp.at[rsl], mtemp_v),
        (ftime.at[rsl], ftime_v),
        (ftemp.at[rsl], ftemp_v),
        (hstg.at[rsl], hstg_v),
        (mstg.at[rsl], mstg_v),
        (ostg.at[rsl], ostg_v),
        (gamt.at[rsl], gamt_v),
        (aamt.at[rsl], aamt_v),
        (htime.at[rsl], htime_v),
        (hconc.at[rsl], hconc_v),
        (mtime2.at[rsl], mtime2_v),
        (mamt.at[rsl], mamt_v),
    ]
    descs = [pltpu.async_copy(s, d, sem) for s, d in copies]
    for d in idx_descs:
      d.wait()

    gathers = []
    for idxb, embb, n in (
        (gidx_v, gemb_v, R * 64 // 128),
        (aidx_v, aemb_v, R * 16 // 128),
        (hidx_v, hemb_v, R * 128 // 128),
        (midx_v, memb_v, R * 32 // 128),
        (oidx_v, oemb_v, R * 16 // 128),
    ):
      for j in range(n):
        gathers.append(_gather_start(
            tbl8, idxb.at[j], embb.at[pl.ds(j * 128, 128)], sem))

    for d in descs:
      d.wait()

    scatters = []

    def put_small(src_v, j, col):
      for g in range(R // L):
        rows = iota + g * L
        if j is None:
          val = _vgather1(src_v, rows)
        else:
          val = _vgather2(src_v, rows, iota * 0 + j)
        _vscatter2(small_v, rows * (SMALL_W // 8) + (col >> 3),
                   iota * 0 + (col & 7), val)

    def put_ones(stg_v, j, buf, nsub, cof, k):
      for g in range(R // L):
        rows = iota + g * L
        t = _vgather2(stg_v, rows, iota * 0 + j)
        cv = cof + j * k + t
        i0 = rows * nsub + lax.shift_right_logical(cv, 3)
        i1 = lax.bitwise_and(cv, 7)
        _vscatter2(buf, i0, i1, ones)
        scatters.append((buf, i0, i1))

    put_small(boil_v, None, 0)
    put_small(ph_v, None, 1)
    put_small(sp_v, None, 2)
    for j in range(4):
      put_ones(mtyp_v, j, small_v, SMALL_W // 8, 3, NUM_MASH_STEP_TYPES)
      put_small(mtime_v, j, 67 + j)
      put_small(mtemp_v, j, 71 + j)
    for j in range(2):
      put_small(ftime_v, j, 75 + j)
      put_small(ftemp_v, j, 77 + j)
    for src_v, w, cof in ((gamt_v, 16, S_GAMT), (aamt_v, 8, S_AAMT),
                          (htime_v, 32, S_HT), (hconc_v, 32, S_HC),
                          (mtime2_v, 16, S_MT), (mamt_v, 16, S_MA)):
      for j in range(w):
        put_small(src_v, j, cof + j)
    for j in range(32):
      put_ones(hstg_v, j, hoh_v, 32, 0, NUM_HOP_STAGE)
    for j in range(16):
      put_ones(mstg_v, j, moh_v, 16, 0, NUM_MISC_STAGE)
    for j in range(8):
      put_ones(ostg_v, j, small_v, SMALL_W // 8, S_OOH, NUM_MO_STAGE)

    for d in gathers:
      d.wait()

    outs = [
        (small_v, small_o.at[pl.ds(row0 * 32, R * 32)]),
        (gemb_v, gemb_o.at[pl.ds(row0 * 64, R * 64)]),
        (aemb_v, aemb_o.at[pl.ds(row0 * 16, R * 16)]),
        (hemb_v, hemb_o.at[pl.ds(row0 * 128, R * 128)]),
        (memb_v, memb_o.at[pl.ds(row0 * 32, R * 32)]),
        (oemb_v, oemb_o.at[pl.ds(row0 * 16, R * 16)]),
        (hoh_v, hoh_o.at[pl.ds(row0 * 32, R * 32)]),
        (moh_v, moh_o.at[pl.ds(row0 * 16, R * 16)]),
    ]
    descs2 = [pltpu.async_copy(s, d, sem) for s, d in outs]
    for d in descs2:
      d.wait()

    for buf, i0, i1 in scatters:
      _vscatter2(buf, i0, i1, zeros)
    return 0

  lax.fori_loop(0, NCHUNK, chunk, 0)


def _concat_body(small_r, gemb_r, aemb_r, hemb_r, memb_r, oemb_r, hoh_r,
                 moh_r, out):
  small = small_r[...].reshape(RB, SMALL_W)
  out[...] = jnp.concatenate([
      small[:, 0:79],
      gemb_r[...].reshape(RB, 512),
      small[:, S_GAMT:S_GAMT + 16],
      aemb_r[...],
      small[:, S_AAMT:S_AAMT + 8],
      hemb_r[...].reshape(RB, 1024),
      hoh_r[...].reshape(RB, 256),
      small[:, S_HT:S_HT + 32],
      small[:, S_HC:S_HC + 32],
      memb_r[...].reshape(RB, 256),
      moh_r[...],
      small[:, S_MT:S_MT + 16],
      small[:, S_MA:S_MA + 16],
      oemb_r[...],
      small[:, S_OOH:S_OOH + 32],
  ], axis=1)


def _assemble(small2, gemb2, aemb2, hemb2, memb2, oemb2, hoh2, moh2):
  def spec(n):
    return pl.BlockSpec((n * RB, 128), lambda i: (i, 0))
  return pl.pallas_call(
      _concat_body,
      grid=(B // RB,),
      in_specs=[spec(2), spec(4), spec(1), spec(8),
                spec(2), spec(1), spec(2), spec(1)],
      out_specs=pl.BlockSpec((RB, OUT_W), lambda i: (i, 0)),
      out_shape=jax.ShapeDtypeStruct((B, OUT_W), jnp.float32),
  )(small2, gemb2, aemb2, hemb2, memb2, oemb2, hoh2, moh2)


@jax.jit
def kernel(boil_time, mash_ph, sparge_temp, mash_step_type_inds,
           mash_step_times, mash_step_avg_temps, ferment_stage_times,
           ferment_stage_temps, grain_core_type_inds, grain_amts,
           adjunct_core_type_inds, adjunct_amts, hop_type_inds,
           hop_stage_type_inds, hop_times, hop_concentrations,
           misc_type_inds, misc_stage_inds, misc_times, misc_amts,
           mo_type_inds, mo_stage_inds, W_grain, W_adjunct, W_hop, W_misc,
           W_mo):
  f32, i32 = jnp.float32, jnp.int32

  tbl8 = jnp.concatenate([
      W_grain.reshape(-1, 8), W_adjunct.reshape(-1, 8),
      W_hop.reshape(-1, 8), W_misc.reshape(-1, 8), W_mo.reshape(-1, 8),
  ], axis=0)

  def sub(idx, base, n):
    e = (base + idx[:, :, None] * n + jnp.arange(n, dtype=i32))
    return e.reshape(-1, 128)

  gidx = sub(grain_core_type_inds, TB_G, 4)
  aidx = sub(adjunct_core_type_inds, TB_A, 2)
  hidx = sub(hop_type_inds, TB_H, 4)
  midx = sub(misc_type_inds, TB_M, 2)
  oidx = sub(mo_type_inds, TB_O, 2)

  mesh = plsc.VectorSubcoreMesh(core_axis_name="c", subcore_axis_name="s",
                                num_cores=NC, num_subcores=NS)
  run = pl.kernel(
      _body,
      out_type=[
          jax.ShapeDtypeStruct((B * 32, 8), f32),
          jax.ShapeDtypeStruct((B * 64, 8), f32),
          jax.ShapeDtypeStruct((B * 16, 8), f32),
          jax.ShapeDtypeStruct((B * 128, 8), f32),
          jax.ShapeDtypeStruct((B * 32, 8), f32),
          jax.ShapeDtypeStruct((B * 16, 8), f32),
          jax.ShapeDtypeStruct((B * 32, 8), f32),
          jax.ShapeDtypeStruct((B * 16, 8), f32),
      ],
      mesh=mesh,
      compiler_params=pltpu.CompilerParams(use_tc_tiling_on_sc=False,
                                           needs_layout_passes=False),
      scratch_types=[
          pltpu.VMEM((R * 64 // 128, 128), i32),
          pltpu.VMEM((R * 16 // 128, 128), i32),
          pltpu.VMEM((R, 128), i32),
          pltpu.VMEM((R * 32 // 128, 128), i32),
          pltpu.VMEM((R * 16 // 128, 128), i32),
          pltpu.VMEM((R * 64, 8), f32),
          pltpu.VMEM((R * 16, 8), f32),
          pltpu.VMEM((R * 128, 8), f32),
          pltpu.VMEM((R * 32, 8), f32),
          pltpu.VMEM((R * 16, 8), f32),
          pltpu.VMEM((R * 32, 8), f32),
          pltpu.VMEM((R * 32, 8), f32),
          pltpu.VMEM((R * 16, 8), f32),
          pltpu.VMEM((R,), f32),
          pltpu.VMEM((R,), f32),
          pltpu.VMEM((R,), f32),
          pltpu.VMEM((R, 4), i32),
          pltpu.VMEM((R, 4), f32),
          pltpu.VMEM((R, 4), f32),
          pltpu.VMEM((R, 2), f32),
          pltpu.VMEM((R, 2), f32),
          pltpu.VMEM((R, 32), i32),
          pltpu.VMEM((R, 16), i32),
          pltpu.VMEM((R, 8), i32),
          pltpu.VMEM((R, 16), f32),
          pltpu.VMEM((R, 8), f32),
          pltpu.VMEM((R, 32), f32),
          pltpu.VMEM((R, 32), f32),
          pltpu.VMEM((R, 16), f32),
          pltpu.VMEM((R, 16), f32),
          pltpu.SemaphoreType.DMA,
      ],
  )
  small_o, gemb_o, aemb_o, hemb_o, memb_o, oemb_o, hoh_o, moh_o = run(
      boil_time, mash_ph, sparge_temp,
      mash_step_type_inds,
      mash_step_times, mash_step_avg_temps,
      ferment_stage_times, ferment_stage_temps,
      gidx, aidx, hidx, midx, oidx, tbl8,
      grain_amts, adjunct_amts, hop_times, hop_concentrations,
      misc_times, misc_amts,
      hop_stage_type_inds, misc_stage_inds, mo_stage_inds,
  )
  return _assemble(
      small_o.reshape(B * 2, 128), gemb_o.reshape(B * 4, 128),
      aemb_o.reshape(B, 128), hemb_o.reshape(B * 8, 128),
      memb_o.reshape(B * 2, 128), oemb_o.reshape(B, 128),
      hoh_o.reshape(B * 2, 128), moh_o.reshape(B, 128),
  )
